# baseline (speedup 1.0000x reference)
import functools

import jax
import jax.numpy as jnp
from jax import lax
from jax.experimental import pallas as pl
from jax.experimental.pallas import tpu as pltpu
from jax.experimental.pallas import tpu_sc as plsc

B = 16384
EMB = 32

_f32 = jnp.float32


@functools.lru_cache(maxsize=None)
def _sc_gather_fn():
    info = plsc.get_sparse_core_info()
    nc, ns = info.num_cores, info.num_subcores
    nw = nc * ns
    bpw = B // nw

    mesh = plsc.VectorSubcoreMesh(
        core_axis_name="c", subcore_axis_name="s", num_cores=nc,
        num_subcores=ns,
    )

    gpw = 2 * bpw

    def body(cidx_i, aidx_i, cidx2_i, aidx2_i, cust_t, art_t, cb_t, ab_t,
             cust_o, art_o, cb_o, ab_o,
             cidx_v, aidx_v, cidx2_v, aidx2_v, bufc, bufa, cbv, abv, sem):
        wid = lax.axis_index("s") * nc + lax.axis_index("c")
        sl = pl.ds(wid * bpw, bpw)
        sl2 = pl.ds(wid * gpw, gpw)
        pltpu.sync_copy(cidx_i.at[sl], cidx_v)
        pltpu.sync_copy(aidx_i.at[sl], aidx_v)
        pltpu.sync_copy(cidx2_i.at[sl2], cidx2_v)
        pltpu.sync_copy(aidx2_i.at[sl2], aidx2_v)

        c0 = pltpu.async_copy(cust_t.at[cidx2_v], bufc, sem)
        c1 = pltpu.async_copy(art_t.at[aidx2_v], bufa, sem)
        c2 = pltpu.async_copy(cb_t.at[cidx_v], cbv, sem)
        c3 = pltpu.async_copy(ab_t.at[aidx_v], abv, sem)
        for c in (c0, c1, c2, c3):
            c.wait()

        pltpu.sync_copy(bufc, cust_o.at[sl2])
        pltpu.sync_copy(bufa, art_o.at[sl2])
        pltpu.sync_copy(cbv, cb_o.at[sl])
        pltpu.sync_copy(abv, ab_o.at[sl])

    return pl.kernel(
        body,
        out_type=(
            jax.ShapeDtypeStruct((2 * B, EMB // 2), _f32),
            jax.ShapeDtypeStruct((2 * B, EMB // 2), _f32),
            jax.ShapeDtypeStruct((B,), _f32),
            jax.ShapeDtypeStruct((B,), _f32),
        ),
        mesh=mesh,
        compiler_params=pltpu.CompilerParams(use_tc_tiling_on_sc=False),
        scratch_types=[
            pltpu.VMEM((bpw,), jnp.int32),
            pltpu.VMEM((bpw,), jnp.int32),
            pltpu.VMEM((gpw,), jnp.int32),
            pltpu.VMEM((gpw,), jnp.int32),
            pltpu.VMEM((gpw, EMB // 2), _f32),
            pltpu.VMEM((gpw, EMB // 2), _f32),
            pltpu.VMEM((bpw,), _f32),
            pltpu.VMEM((bpw,), _f32),
            pltpu.SemaphoreType.DMA,
        ],
    )


def _sigmoid(x):
    return 1.0 / (1.0 + jnp.exp(-x))


TB = 2048

NUM_AGE = 100
NUM_IDXGROUP = 10
NUM_GARMENT = 21


def _onehot(idx_blk, n):
    classes = lax.broadcasted_iota(jnp.int32, (idx_blk.shape[0], n), 1)
    return jnp.where(classes == idx_blk, 1.0, 0.0).astype(_f32)


def _dot(a, b):
    return jnp.dot(a, b, preferred_element_type=_f32)


def _tc_body(cust_r, art_r, cb_r, ab_r, age_r, idxg_r, gar_r,
             age_t, idxg_t, gar_t, wc_r, bc_r, wa_r, ba_r, out_r):
    wc = wc_r[...]
    wa = wa_r[...]
    age_proj = _dot(age_t[...], wc[EMB:2 * EMB])
    idxg_proj = _dot(idxg_t[...], wa[EMB:2 * EMB])
    gar_proj = _dot(gar_t[...], wa[2 * EMB:3 * EMB])

    cm = _sigmoid(
        _dot(cust_r[...], wc[0:EMB])
        + _dot(_onehot(age_r[...], NUM_AGE), age_proj)
        + bc_r[...]
    )
    am = _sigmoid(
        _dot(art_r[...], wa[0:EMB])
        + _dot(_onehot(idxg_r[...], NUM_IDXGROUP), idxg_proj)
        + _dot(_onehot(gar_r[...], NUM_GARMENT), gar_proj)
        + ba_r[...]
    )
    x = jnp.sum(cm * am, axis=1, keepdims=True) + cb_r[...] + ab_r[...]
    out_r[...] = _sigmoid(x)


def _tc_dense(cust_rows, art_rows, cb, ab, age_i, idxg_i, gar_i,
              age_t, idxg_t, gar_t, w_cust, b_cust, w_art, b_art):
    n_blocks = B // TB
    row_spec = pl.BlockSpec((TB, EMB), lambda i: (i, 0))
    col_spec = pl.BlockSpec((TB, 1), lambda i: (i, 0))
    full = lambda shape: pl.BlockSpec(shape, lambda i: (0, 0))
    return pl.pallas_call(
        _tc_body,
        grid=(n_blocks,),
        in_specs=[
            row_spec, row_spec, col_spec, col_spec,
            col_spec, col_spec, col_spec,
            full((NUM_AGE, EMB)), full((NUM_IDXGROUP, EMB)),
            full((NUM_GARMENT, EMB)),
            full((2 * EMB, EMB)), full((1, EMB)),
            full((3 * EMB, EMB)), full((1, EMB)),
        ],
        out_specs=col_spec,
        out_shape=jax.ShapeDtypeStruct((B, 1), _f32),
    )(cust_rows, art_rows, cb, ab, age_i, idxg_i, gar_i,
      age_t, idxg_t, gar_t, w_cust, b_cust, w_art, b_art)


def kernel(row, customer_embed, art_embed, customer_bias, article_bias,
           age_embed, indexgroup_embed, garmentgroup_embed,
           W_art, b_art, W_cust, b_cust):
    row = row.astype(jnp.int32)
    cust = row[:, 0]
    art = row[:, 1]
    age = jnp.where(row[:, 2] < 0, 36, row[:, 2]) - 1
    gar = row[:, 3] - 1001
    idxg = row[:, 4] - 1

    two = jnp.arange(2, dtype=jnp.int32)
    cust2 = (cust[:, None] * 2 + two).reshape(-1)
    art2 = (art[:, None] * 2 + two).reshape(-1)

    ce128 = lax.optimization_barrier(customer_embed.reshape(-1, 128))
    ae128 = lax.optimization_barrier(art_embed.reshape(-1, 128))

    cust_rows, art_rows, cb, ab = _sc_gather_fn()(
        cust, art, cust2, art2,
        ce128.reshape(-1, EMB // 2),
        ae128.reshape(-1, EMB // 2),
        customer_bias.reshape(-1), article_bias.reshape(-1),
    )
    return _tc_dense(
        cust_rows.reshape(B, EMB), art_rows.reshape(B, EMB),
        cb.reshape(B, 1), ab.reshape(B, 1),
        age.reshape(B, 1), idxg.reshape(B, 1), gar.reshape(B, 1),
        age_embed, indexgroup_embed, garmentgroup_embed,
        W_cust, b_cust.reshape(1, EMB), W_art, b_art.reshape(1, EMB),
    )


# speedup vs baseline: 1.2598x; 1.2598x over previous
import functools

import jax
import jax.numpy as jnp
from jax import lax
from jax.experimental import pallas as pl
from jax.experimental.pallas import tpu as pltpu
from jax.experimental.pallas import tpu_sc as plsc

B = 16384
EMB = 32

_f32 = jnp.float32

CUST_N = 1_000_000
ART_N = 100_000
CB_ROWS = 7813
AB_ROWS = 782

PNB = 8192


def _project_body(ct_r, w_r, out_r):
    x = ct_r[...]
    w = w_r[...]
    zb = lax.dot_general(x, w, (((0,), (0,)), ((), ())),
                         preferred_element_type=_f32)
    groups = []
    for s in range(PNB // 512):
        groups.append(jnp.concatenate(
            [zb[512 * s + 128 * q: 512 * s + 128 * (q + 1), :]
             for q in range(4)], axis=1))
    out_r[...] = jnp.concatenate(groups, axis=0)


def _project(table_t, w, n):
    grid = -(-n // PNB)
    return pl.pallas_call(
        _project_body,
        grid=(grid,),
        in_specs=[
            pl.BlockSpec((EMB, PNB), lambda i: (0, i)),
            pl.BlockSpec((EMB, EMB), lambda i: (0, 0)),
        ],
        out_specs=pl.BlockSpec((PNB // 4, 128), lambda i: (i, 0)),
        out_shape=jax.ShapeDtypeStruct((grid * (PNB // 4), 128), _f32),
    )(table_t, w)


@functools.lru_cache(maxsize=None)
def _sc_gather_fn():
    info = plsc.get_sparse_core_info()
    nc, ns = info.num_cores, info.num_subcores
    nw = nc * ns
    bpw = B // nw
    ch = 128
    nch = bpw // ch

    mesh = plsc.VectorSubcoreMesh(
        core_axis_name="c", subcore_axis_name="s", num_cores=nc,
        num_subcores=ns,
    )

    def body(ic_i, ia_i, icb_i, iab_i, cust_t, art_t, cb_t, ab_t,
             cust_o, art_o, cb_o, ab_o,
             icv, iav, icbv, iabv, bufc, bufa, bufcb, bufab, sem):
        wid = lax.axis_index("s") * nc + lax.axis_index("c")
        for k in range(nch):
            sl = pl.ds(wid * bpw + k * ch, ch)
            pltpu.sync_copy(ic_i.at[sl], icv)
            pltpu.sync_copy(ia_i.at[sl], iav)
            pltpu.sync_copy(icb_i.at[sl], icbv)
            pltpu.sync_copy(iab_i.at[sl], iabv)

            c0 = pltpu.async_copy(cust_t.at[icv], bufc, sem)
            c1 = pltpu.async_copy(art_t.at[iav], bufa, sem)
            c2 = pltpu.async_copy(cb_t.at[icbv], bufcb, sem)
            c3 = pltpu.async_copy(ab_t.at[iabv], bufab, sem)
            for c in (c0, c1, c2, c3):
                c.wait()

            pltpu.sync_copy(bufc, cust_o.at[sl])
            pltpu.sync_copy(bufa, art_o.at[sl])
            pltpu.sync_copy(bufcb, cb_o.at[sl])
            pltpu.sync_copy(bufab, ab_o.at[sl])

    return pl.kernel(
        body,
        out_type=(
            jax.ShapeDtypeStruct((B, 128), _f32),
            jax.ShapeDtypeStruct((B, 128), _f32),
            jax.ShapeDtypeStruct((B, 128), _f32),
            jax.ShapeDtypeStruct((B, 128), _f32),
        ),
        mesh=mesh,
        scratch_types=[
            pltpu.VMEM((128,), jnp.int32),
            pltpu.VMEM((128,), jnp.int32),
            pltpu.VMEM((128,), jnp.int32),
            pltpu.VMEM((128,), jnp.int32),
            pltpu.VMEM((128, 128), _f32),
            pltpu.VMEM((128, 128), _f32),
            pltpu.VMEM((128, 128), _f32),
            pltpu.VMEM((128, 128), _f32),
            pltpu.SemaphoreType.DMA,
        ],
    )


def _sigmoid(x):
    return 1.0 / (1.0 + jnp.exp(-x))


TB = 2048

NUM_AGE = 100
NUM_IDXGROUP = 10
NUM_GARMENT = 21


def _onehot(idx_blk, n):
    classes = lax.broadcasted_iota(jnp.int32, (idx_blk.shape[0], n), 1)
    return jnp.where(classes == idx_blk, 1.0, 0.0).astype(_f32)


def _dot(a, b):
    return jnp.dot(a, b, preferred_element_type=_f32)


def _quarter_select(g, q_blk):
    out = jnp.zeros((g.shape[0], EMB), _f32)
    for q in range(4):
        out = out + jnp.where(q_blk == q, g[:, q * EMB:(q + 1) * EMB], 0.0)
    return out


def _lane_select(g, l_blk):
    lanes = lax.broadcasted_iota(jnp.int32, g.shape, 1)
    return jnp.sum(jnp.where(lanes == l_blk, g, 0.0), axis=1, keepdims=True)


def _tc_body(zc_g, za_g, cbg_r, abg_r, cq_r, aq_r, cbl_r, abl_r,
             age_r, idxg_r, gar_r,
             age_t, idxg_t, gar_t, wc_r, bc_r, wa_r, ba_r, out_r):
    wc = wc_r[...]
    wa = wa_r[...]
    zc = _quarter_select(zc_g[...], cq_r[...])
    za = _quarter_select(za_g[...], aq_r[...])
    cb = _lane_select(cbg_r[...], cbl_r[...])
    ab = _lane_select(abg_r[...], abl_r[...])

    age_proj = _dot(age_t[...], wc[EMB:2 * EMB])
    idxg_proj = _dot(idxg_t[...], wa[EMB:2 * EMB])
    gar_proj = _dot(gar_t[...], wa[2 * EMB:3 * EMB])

    cm = _sigmoid(
        zc + _dot(_onehot(age_r[...], NUM_AGE), age_proj) + bc_r[...]
    )
    am = _sigmoid(
        za
        + _dot(_onehot(idxg_r[...], NUM_IDXGROUP), idxg_proj)
        + _dot(_onehot(gar_r[...], NUM_GARMENT), gar_proj)
        + ba_r[...]
    )
    x = jnp.sum(cm * am, axis=1, keepdims=True) + cb + ab
    out_r[...] = _sigmoid(x)


def _tc_dense(zc_g, za_g, cb_g, ab_g, cq, aq, cbl, abl,
              age_i, idxg_i, gar_i,
              age_t, idxg_t, gar_t, w_cust, b_cust, w_art, b_art):
    n_blocks = B // TB
    g_spec = pl.BlockSpec((TB, 128), lambda i: (i, 0))
    col_spec = pl.BlockSpec((TB, 1), lambda i: (i, 0))
    full = lambda shape: pl.BlockSpec(shape, lambda i: (0, 0))
    return pl.pallas_call(
        _tc_body,
        grid=(n_blocks,),
        in_specs=[
            g_spec, g_spec, g_spec, g_spec,
            col_spec, col_spec, col_spec, col_spec,
            col_spec, col_spec, col_spec,
            full((NUM_AGE, EMB)), full((NUM_IDXGROUP, EMB)),
            full((NUM_GARMENT, EMB)),
            full((2 * EMB, EMB)), full((1, EMB)),
            full((3 * EMB, EMB)), full((1, EMB)),
        ],
        out_specs=col_spec,
        out_shape=jax.ShapeDtypeStruct((B, 1), _f32),
    )(zc_g, za_g, cb_g, ab_g, cq, aq, cbl, abl,
      age_i, idxg_i, gar_i,
      age_t, idxg_t, gar_t, w_cust, b_cust, w_art, b_art)


def kernel(row, customer_embed, art_embed, customer_bias, article_bias,
           age_embed, indexgroup_embed, garmentgroup_embed,
           W_art, b_art, W_cust, b_cust):
    row = row.astype(jnp.int32)
    cust = row[:, 0]
    art = row[:, 1]
    age = jnp.where(row[:, 2] < 0, 36, row[:, 2]) - 1
    gar = row[:, 3] - 1001
    idxg = row[:, 4] - 1

    zc_tbl = _project(customer_embed.T, W_cust[0:EMB], CUST_N)
    za_tbl = _project(art_embed.T, W_art[0:EMB], ART_N)

    cb_tbl = jnp.pad(customer_bias.reshape(-1),
                     (0, CB_ROWS * 128 - CUST_N)).reshape(-1, 128)
    ab_tbl = jnp.pad(article_bias.reshape(-1),
                     (0, AB_ROWS * 128 - ART_N)).reshape(-1, 128)

    ic = 128 * (cust // 512) + cust % 128
    ia = 128 * (art // 512) + art % 128

    zc_g, za_g, cb_g, ab_g = _sc_gather_fn()(
        ic, ia, cust // 128, art // 128,
        zc_tbl, za_tbl, cb_tbl, ab_tbl,
    )
    return _tc_dense(
        zc_g, za_g, cb_g, ab_g,
        ((cust % 512) // 128).reshape(B, 1),
        ((art % 512) // 128).reshape(B, 1),
        (cust % 128).reshape(B, 1), (art % 128).reshape(B, 1),
        age.reshape(B, 1), idxg.reshape(B, 1), gar.reshape(B, 1),
        age_embed, indexgroup_embed, garmentgroup_embed,
        W_cust, b_cust.reshape(1, EMB), W_art, b_art.reshape(1, EMB),
    )


# speedup vs baseline: 1.2602x; 1.0003x over previous
import functools

import jax
import jax.numpy as jnp
from jax import lax
from jax.experimental import pallas as pl
from jax.experimental.pallas import tpu as pltpu
from jax.experimental.pallas import tpu_sc as plsc

B = 16384
EMB = 32

_f32 = jnp.float32

CUST_N = 1_000_000
ART_N = 100_000
CB_ROWS = 7813
AB_ROWS = 782

PNB = 8192


def _project_body(ct_r, w_r, out_r):
    x = ct_r[...]
    w = w_r[...]
    zb = lax.dot_general(x, w, (((0,), (0,)), ((), ())),
                         preferred_element_type=_f32)
    groups = []
    for s in range(PNB // 512):
        groups.append(jnp.concatenate(
            [zb[512 * s + 128 * q: 512 * s + 128 * (q + 1), :]
             for q in range(4)], axis=1))
    out_r[...] = jnp.concatenate(groups, axis=0)


def _project(table_t, w, n):
    grid = -(-n // PNB)
    return pl.pallas_call(
        _project_body,
        grid=(grid,),
        in_specs=[
            pl.BlockSpec((EMB, PNB), lambda i: (0, i)),
            pl.BlockSpec((EMB, EMB), lambda i: (0, 0)),
        ],
        out_specs=pl.BlockSpec((PNB // 4, 128), lambda i: (i, 0)),
        out_shape=jax.ShapeDtypeStruct((grid * (PNB // 4), 128), _f32),
        compiler_params=pltpu.CompilerParams(
            dimension_semantics=("parallel",)),
    )(table_t, w)


@functools.lru_cache(maxsize=None)
def _sc_gather_fn():
    info = plsc.get_sparse_core_info()
    nc, ns = info.num_cores, info.num_subcores
    nw = nc * ns
    bpw = B // nw
    ch = 128
    nch = bpw // ch

    mesh = plsc.VectorSubcoreMesh(
        core_axis_name="c", subcore_axis_name="s", num_cores=nc,
        num_subcores=ns,
    )

    def body(ic_i, ia_i, icb_i, iab_i, cust_t, art_t, cb_t, ab_t,
             cust_o, art_o, cb_o, ab_o,
             icv, iav, icbv, iabv, bufc, bufa, bufcb, bufab, sem):
        wid = lax.axis_index("s") * nc + lax.axis_index("c")
        for k in range(nch):
            sl = pl.ds(wid * bpw + k * ch, ch)
            pltpu.sync_copy(ic_i.at[sl], icv)
            pltpu.sync_copy(ia_i.at[sl], iav)
            pltpu.sync_copy(icb_i.at[sl], icbv)
            pltpu.sync_copy(iab_i.at[sl], iabv)

            c0 = pltpu.async_copy(cust_t.at[icv], bufc, sem)
            c1 = pltpu.async_copy(art_t.at[iav], bufa, sem)
            c2 = pltpu.async_copy(cb_t.at[icbv], bufcb, sem)
            c3 = pltpu.async_copy(ab_t.at[iabv], bufab, sem)
            for c in (c0, c1, c2, c3):
                c.wait()

            pltpu.sync_copy(bufc, cust_o.at[sl])
            pltpu.sync_copy(bufa, art_o.at[sl])
            pltpu.sync_copy(bufcb, cb_o.at[sl])
            pltpu.sync_copy(bufab, ab_o.at[sl])

    return pl.kernel(
        body,
        out_type=(
            jax.ShapeDtypeStruct((B, 128), _f32),
            jax.ShapeDtypeStruct((B, 128), _f32),
            jax.ShapeDtypeStruct((B, 128), _f32),
            jax.ShapeDtypeStruct((B, 128), _f32),
        ),
        mesh=mesh,
        scratch_types=[
            pltpu.VMEM((128,), jnp.int32),
            pltpu.VMEM((128,), jnp.int32),
            pltpu.VMEM((128,), jnp.int32),
            pltpu.VMEM((128,), jnp.int32),
            pltpu.VMEM((128, 128), _f32),
            pltpu.VMEM((128, 128), _f32),
            pltpu.VMEM((128, 128), _f32),
            pltpu.VMEM((128, 128), _f32),
            pltpu.SemaphoreType.DMA,
        ],
    )


def _sigmoid(x):
    return 1.0 / (1.0 + jnp.exp(-x))


TB = 2048

NUM_AGE = 100
NUM_IDXGROUP = 10
NUM_GARMENT = 21


def _onehot(idx_blk, n):
    classes = lax.broadcasted_iota(jnp.int32, (idx_blk.shape[0], n), 1)
    return jnp.where(classes == idx_blk, 1.0, 0.0).astype(_f32)


def _dot(a, b):
    return jnp.dot(a, b, preferred_element_type=_f32)


def _quarter_select(g, q_blk):
    out = jnp.zeros((g.shape[0], EMB), _f32)
    for q in range(4):
        out = out + jnp.where(q_blk == q, g[:, q * EMB:(q + 1) * EMB], 0.0)
    return out


def _lane_select(g, l_blk):
    lanes = lax.broadcasted_iota(jnp.int32, g.shape, 1)
    return jnp.sum(jnp.where(lanes == l_blk, g, 0.0), axis=1, keepdims=True)


def _tc_body(zc_g, za_g, cbg_r, abg_r, cq_r, aq_r, cbl_r, abl_r,
             age_r, idxg_r, gar_r,
             age_t, idxg_t, gar_t, wc_r, bc_r, wa_r, ba_r, out_r):
    wc = wc_r[...]
    wa = wa_r[...]
    zc = _quarter_select(zc_g[...], cq_r[...])
    za = _quarter_select(za_g[...], aq_r[...])
    cb = _lane_select(cbg_r[...], cbl_r[...])
    ab = _lane_select(abg_r[...], abl_r[...])

    age_proj = _dot(age_t[...], wc[EMB:2 * EMB])
    idxg_proj = _dot(idxg_t[...], wa[EMB:2 * EMB])
    gar_proj = _dot(gar_t[...], wa[2 * EMB:3 * EMB])

    cm = _sigmoid(
        zc + _dot(_onehot(age_r[...], NUM_AGE), age_proj) + bc_r[...]
    )
    am = _sigmoid(
        za
        + _dot(_onehot(idxg_r[...], NUM_IDXGROUP), idxg_proj)
        + _dot(_onehot(gar_r[...], NUM_GARMENT), gar_proj)
        + ba_r[...]
    )
    x = jnp.sum(cm * am, axis=1, keepdims=True) + cb + ab
    out_r[...] = _sigmoid(x)


def _tc_dense(zc_g, za_g, cb_g, ab_g, cq, aq, cbl, abl,
              age_i, idxg_i, gar_i,
              age_t, idxg_t, gar_t, w_cust, b_cust, w_art, b_art):
    n_blocks = B // TB
    g_spec = pl.BlockSpec((TB, 128), lambda i: (i, 0))
    col_spec = pl.BlockSpec((TB, 1), lambda i: (i, 0))
    full = lambda shape: pl.BlockSpec(shape, lambda i: (0, 0))
    return pl.pallas_call(
        _tc_body,
        grid=(n_blocks,),
        in_specs=[
            g_spec, g_spec, g_spec, g_spec,
            col_spec, col_spec, col_spec, col_spec,
            col_spec, col_spec, col_spec,
            full((NUM_AGE, EMB)), full((NUM_IDXGROUP, EMB)),
            full((NUM_GARMENT, EMB)),
            full((2 * EMB, EMB)), full((1, EMB)),
            full((3 * EMB, EMB)), full((1, EMB)),
        ],
        out_specs=col_spec,
        out_shape=jax.ShapeDtypeStruct((B, 1), _f32),
        compiler_params=pltpu.CompilerParams(
            dimension_semantics=("parallel",)),
    )(zc_g, za_g, cb_g, ab_g, cq, aq, cbl, abl,
      age_i, idxg_i, gar_i,
      age_t, idxg_t, gar_t, w_cust, b_cust, w_art, b_art)


def kernel(row, customer_embed, art_embed, customer_bias, article_bias,
           age_embed, indexgroup_embed, garmentgroup_embed,
           W_art, b_art, W_cust, b_cust):
    row = row.astype(jnp.int32)
    cust = row[:, 0]
    art = row[:, 1]
    age = jnp.where(row[:, 2] < 0, 36, row[:, 2]) - 1
    gar = row[:, 3] - 1001
    idxg = row[:, 4] - 1

    zc_tbl = _project(customer_embed.T, W_cust[0:EMB], CUST_N)
    za_tbl = _project(art_embed.T, W_art[0:EMB], ART_N)

    cb_tbl = jnp.pad(customer_bias,
                     ((0, CB_ROWS * 128 - CUST_N), (0, 0))).reshape(-1, 128)
    ab_tbl = jnp.pad(article_bias,
                     ((0, AB_ROWS * 128 - ART_N), (0, 0))).reshape(-1, 128)

    ic = 128 * (cust // 512) + cust % 128
    ia = 128 * (art // 512) + art % 128

    zc_g, za_g, cb_g, ab_g = _sc_gather_fn()(
        ic, ia, cust // 128, art // 128,
        zc_tbl, za_tbl, cb_tbl, ab_tbl,
    )
    return _tc_dense(
        zc_g, za_g, cb_g, ab_g,
        ((cust % 512) // 128).reshape(B, 1),
        ((art % 512) // 128).reshape(B, 1),
        (cust % 128).reshape(B, 1), (art % 128).reshape(B, 1),
        age.reshape(B, 1), idxg.reshape(B, 1), gar.reshape(B, 1),
        age_embed, indexgroup_embed, garmentgroup_embed,
        W_cust, b_cust.reshape(1, EMB), W_art, b_art.reshape(1, EMB),
    )


# speedup vs baseline: 1.6661x; 1.3221x over previous
import functools

import jax
import jax.numpy as jnp
from jax import lax
from jax.experimental import pallas as pl
from jax.experimental.pallas import tpu as pltpu
from jax.experimental.pallas import tpu_sc as plsc

B = 16384
EMB = 32

_f32 = jnp.float32

CUST_N = 1_000_000
ART_N = 100_000
CB_ROWS = 7813
AB_ROWS = 782

PNB = 8192


def _project_body(ct_r, w_r, out_r):
    x = ct_r[...]
    wbd = w_r[...]
    groups = []
    for s in range(PNB // 512):
        xs = jnp.concatenate(
            [x[:, 512 * s + 128 * q: 512 * s + 128 * (q + 1)]
             for q in range(4)], axis=0)
        groups.append(lax.dot_general(xs, wbd, (((0,), (0,)), ((), ())),
                                      preferred_element_type=_f32))
    out_r[...] = jnp.concatenate(groups, axis=0)


def _project(table_t, wbd, n):
    grid = -(-n // PNB)
    return pl.pallas_call(
        _project_body,
        grid=(grid,),
        in_specs=[
            pl.BlockSpec((EMB, PNB), lambda i: (0, i)),
            pl.BlockSpec((128, 128), lambda i: (0, 0)),
        ],
        out_specs=pl.BlockSpec((PNB // 4, 128), lambda i: (i, 0)),
        out_shape=jax.ShapeDtypeStruct((grid * (PNB // 4), 128), _f32),
        compiler_params=pltpu.CompilerParams(
            dimension_semantics=("parallel",)),
    )(table_t, wbd)


@functools.lru_cache(maxsize=None)
def _sc_gather_fn():
    info = plsc.get_sparse_core_info()
    nc, ns = info.num_cores, info.num_subcores
    nw = nc * ns
    bpw = B // nw
    ch = 128
    nch = bpw // ch

    mesh = plsc.VectorSubcoreMesh(
        core_axis_name="c", subcore_axis_name="s", num_cores=nc,
        num_subcores=ns,
    )

    def body(ic_i, ia_i, icb_i, iab_i, cust_t, art_t, cb_t, ab_t,
             cust_o, art_o, cb_o, ab_o,
             icv, iav, icbv, iabv, bufc, bufa, bufcb, bufab, sem):
        wid = lax.axis_index("s") * nc + lax.axis_index("c")
        for k in range(nch):
            sl = pl.ds(wid * bpw + k * ch, ch)
            pltpu.sync_copy(ic_i.at[sl], icv)
            pltpu.sync_copy(ia_i.at[sl], iav)
            pltpu.sync_copy(icb_i.at[sl], icbv)
            pltpu.sync_copy(iab_i.at[sl], iabv)

            c0 = pltpu.async_copy(cust_t.at[icv], bufc, sem)
            c1 = pltpu.async_copy(art_t.at[iav], bufa, sem)
            c2 = pltpu.async_copy(cb_t.at[icbv], bufcb, sem)
            c3 = pltpu.async_copy(ab_t.at[iabv], bufab, sem)
            for c in (c0, c1, c2, c3):
                c.wait()

            pltpu.sync_copy(bufc, cust_o.at[sl])
            pltpu.sync_copy(bufa, art_o.at[sl])
            pltpu.sync_copy(bufcb, cb_o.at[sl])
            pltpu.sync_copy(bufab, ab_o.at[sl])

    return pl.kernel(
        body,
        out_type=(
            jax.ShapeDtypeStruct((B, 128), _f32),
            jax.ShapeDtypeStruct((B, 128), _f32),
            jax.ShapeDtypeStruct((B, 128), _f32),
            jax.ShapeDtypeStruct((B, 128), _f32),
        ),
        mesh=mesh,
        scratch_types=[
            pltpu.VMEM((128,), jnp.int32),
            pltpu.VMEM((128,), jnp.int32),
            pltpu.VMEM((128,), jnp.int32),
            pltpu.VMEM((128,), jnp.int32),
            pltpu.VMEM((128, 128), _f32),
            pltpu.VMEM((128, 128), _f32),
            pltpu.VMEM((128, 128), _f32),
            pltpu.VMEM((128, 128), _f32),
            pltpu.SemaphoreType.DMA,
        ],
    )


def _sigmoid(x):
    return 1.0 / (1.0 + jnp.exp(-x))


TB = 2048

NUM_AGE = 100
NUM_IDXGROUP = 10
NUM_GARMENT = 21


def _onehot(idx_blk, n):
    classes = lax.broadcasted_iota(jnp.int32, (idx_blk.shape[0], n), 1)
    return jnp.where(classes == idx_blk, 1.0, 0.0).astype(_f32)


def _dot(a, b):
    return jnp.dot(a, b, preferred_element_type=_f32)


def _quarter_select(g, q_blk):
    out = jnp.zeros((g.shape[0], EMB), _f32)
    for q in range(4):
        out = out + jnp.where(q_blk == q, g[:, q * EMB:(q + 1) * EMB], 0.0)
    return out


def _lane_select(g, l_blk):
    lanes = lax.broadcasted_iota(jnp.int32, g.shape, 1)
    return jnp.sum(jnp.where(lanes == l_blk, g, 0.0), axis=1, keepdims=True)


def _tc_body(zc_g, za_g, cbg_r, abg_r, cq_r, aq_r, cbl_r, abl_r,
             age_r, idxg_r, gar_r,
             age_t, idxg_t, gar_t, wc_r, bc_r, wa_r, ba_r, out_r):
    wc = wc_r[...]
    wa = wa_r[...]
    zc = _quarter_select(zc_g[...], cq_r[...])
    za = _quarter_select(za_g[...], aq_r[...])
    cb = _lane_select(cbg_r[...], cbl_r[...])
    ab = _lane_select(abg_r[...], abl_r[...])

    age_proj = _dot(age_t[...], wc[EMB:2 * EMB])
    idxg_proj = _dot(idxg_t[...], wa[EMB:2 * EMB])
    gar_proj = _dot(gar_t[...], wa[2 * EMB:3 * EMB])

    cm = _sigmoid(
        zc + _dot(_onehot(age_r[...], NUM_AGE), age_proj) + bc_r[...]
    )
    am = _sigmoid(
        za
        + _dot(_onehot(idxg_r[...], NUM_IDXGROUP), idxg_proj)
        + _dot(_onehot(gar_r[...], NUM_GARMENT), gar_proj)
        + ba_r[...]
    )
    x = jnp.sum(cm * am, axis=1, keepdims=True) + cb + ab
    out_r[...] = _sigmoid(x)


def _tc_dense(zc_g, za_g, cb_g, ab_g, cq, aq, cbl, abl,
              age_i, idxg_i, gar_i,
              age_t, idxg_t, gar_t, w_cust, b_cust, w_art, b_art):
    n_blocks = B // TB
    g_spec = pl.BlockSpec((TB, 128), lambda i: (i, 0))
    col_spec = pl.BlockSpec((TB, 1), lambda i: (i, 0))
    full = lambda shape: pl.BlockSpec(shape, lambda i: (0, 0))
    return pl.pallas_call(
        _tc_body,
        grid=(n_blocks,),
        in_specs=[
            g_spec, g_spec, g_spec, g_spec,
            col_spec, col_spec, col_spec, col_spec,
            col_spec, col_spec, col_spec,
            full((NUM_AGE, EMB)), full((NUM_IDXGROUP, EMB)),
            full((NUM_GARMENT, EMB)),
            full((2 * EMB, EMB)), full((1, EMB)),
            full((3 * EMB, EMB)), full((1, EMB)),
        ],
        out_specs=col_spec,
        out_shape=jax.ShapeDtypeStruct((B, 1), _f32),
        compiler_params=pltpu.CompilerParams(
            dimension_semantics=("parallel",)),
    )(zc_g, za_g, cb_g, ab_g, cq, aq, cbl, abl,
      age_i, idxg_i, gar_i,
      age_t, idxg_t, gar_t, w_cust, b_cust, w_art, b_art)


def kernel(row, customer_embed, art_embed, customer_bias, article_bias,
           age_embed, indexgroup_embed, garmentgroup_embed,
           W_art, b_art, W_cust, b_cust):
    row = row.astype(jnp.int32)
    cust = row[:, 0]
    art = row[:, 1]
    age = jnp.where(row[:, 2] < 0, 36, row[:, 2]) - 1
    gar = row[:, 3] - 1001
    idxg = row[:, 4] - 1

    eye4 = jnp.eye(4, dtype=_f32)
    wc_bd = jnp.kron(eye4, W_cust[0:EMB])
    wa_bd = jnp.kron(eye4, W_art[0:EMB])
    zc_tbl = _project(customer_embed.T, wc_bd, CUST_N)
    za_tbl = _project(art_embed.T, wa_bd, ART_N)

    cb_tbl = jnp.pad(customer_bias,
                     ((0, CB_ROWS * 128 - CUST_N), (0, 0))).reshape(-1, 128)
    ab_tbl = jnp.pad(article_bias,
                     ((0, AB_ROWS * 128 - ART_N), (0, 0))).reshape(-1, 128)

    ic = 128 * (cust // 512) + cust % 128
    ia = 128 * (art // 512) + art % 128

    zc_g, za_g, cb_g, ab_g = _sc_gather_fn()(
        ic, ia, cust // 128, art // 128,
        zc_tbl, za_tbl, cb_tbl, ab_tbl,
    )
    return _tc_dense(
        zc_g, za_g, cb_g, ab_g,
        ((cust % 512) // 128).reshape(B, 1),
        ((art % 512) // 128).reshape(B, 1),
        (cust % 128).reshape(B, 1), (art % 128).reshape(B, 1),
        age.reshape(B, 1), idxg.reshape(B, 1), gar.reshape(B, 1),
        age_embed, indexgroup_embed, garmentgroup_embed,
        W_cust, b_cust.reshape(1, EMB), W_art, b_art.reshape(1, EMB),
    )
